# baseline (speedup 1.0000x reference)
import functools

import jax
import jax.numpy as jnp
from jax import lax
from jax.experimental import pallas as pl
from jax.experimental.pallas import tpu as pltpu
from jax.experimental.pallas import tpu_sc as plsc

_EPS = 1e-5
_BN = 1.0 / (1.0 + _EPS) ** 0.5


def _lrelu(x, slope):
    return jnp.where(x > 0, x, slope * x)


def _in_mlp_body(h_ref, w_ref, b_ref, o_ref):
    x = _lrelu(h_ref[...] * _BN, 0.2)
    y = jnp.dot(x, w_ref[...], preferred_element_type=jnp.float32) + b_ref[...]
    o_ref[...] = _lrelu(y * _BN, 0.2)


def _tc_in_mlp(h, W, b):
    N, D = h.shape
    BLK = 1000
    return pl.pallas_call(
        _in_mlp_body,
        grid=(N // BLK,),
        in_specs=[
            pl.BlockSpec((BLK, D), lambda i: (i, 0)),
            pl.BlockSpec((D, D), lambda i: (0, 0)),
            pl.BlockSpec((1, D), lambda i: (0, 0)),
        ],
        out_specs=pl.BlockSpec((BLK, D), lambda i: (i, 0)),
        out_shape=jax.ShapeDtypeStruct((N, D), jnp.float32),
    )(h, W, b.reshape(1, D))


def _edge_mlp_body(he_ref, w_ref, b_ref, o_ref):
    x = _lrelu(he_ref[...] * _BN, 0.2)[:, : w_ref.shape[0]]
    y = jnp.dot(x, w_ref[...], preferred_element_type=jnp.float32) + b_ref[...]
    o_ref[...] = _lrelu(y * _BN, 0.2)


def _tc_edge_mlp(h_e, W, b):
    E, _ = h_e.shape
    DE, D = W.shape
    BLK = 4096 if E % 4096 == 0 else 4000
    return pl.pallas_call(
        _edge_mlp_body,
        grid=(E // BLK,),
        in_specs=[
            pl.BlockSpec((BLK, h_e.shape[1]), lambda i: (i, 0)),
            pl.BlockSpec((DE, D), lambda i: (0, 0)),
            pl.BlockSpec((1, D), lambda i: (0, 0)),
        ],
        out_specs=pl.BlockSpec((BLK, D), lambda i: (i, 0)),
        out_shape=jax.ShapeDtypeStruct((E, D), jnp.float32),
    )(h_e, W, b.reshape(1, D))


def _out_mlp_body(u_ref, h_ref, wl_ref, wo_ref, o_ref):
    x = _lrelu(u_ref[...] * _BN, 0.1)
    y = jnp.dot(x, wl_ref[...], preferred_element_type=jnp.float32)
    y = _lrelu(y * _BN, 0.1)
    z = jnp.dot(y, wo_ref[...], preferred_element_type=jnp.float32)
    o_ref[...] = z + h_ref[...]


def _tc_out_mlp(upd, h, W_lin, W_out):
    N, RD = upd.shape
    D = W_out.shape[1]
    BLK = 1000
    return pl.pallas_call(
        _out_mlp_body,
        grid=(N // BLK,),
        in_specs=[
            pl.BlockSpec((BLK, RD), lambda i: (i, 0)),
            pl.BlockSpec((BLK, D), lambda i: (i, 0)),
            pl.BlockSpec((RD, D), lambda i: (0, 0)),
            pl.BlockSpec((D, D), lambda i: (0, 0)),
        ],
        out_specs=pl.BlockSpec((BLK, D), lambda i: (i, 0)),
        out_shape=jax.ShapeDtypeStruct((N, D), jnp.float32),
    )(upd, h, W_lin, W_out)


_NC, _NS = 2, 16


@functools.lru_cache(maxsize=None)
def _make_sc_kernels(N, E, D, R):
    NR = N * R
    BR = 4096
    NB = -(-NR // BR)
    NBC = NB // _NC
    TRASH = BR
    EPT = E // _NS
    S = 4000
    NCH = EPT // S
    GC = 128
    CAP = ((EPT + 143) // GC + 1) * GC
    SEGS = _NC * _NS * NBC
    TOTSEG = SEGS * CAP
    OFF_L, OFF_S, OFF_E, OFF_C = 0, TOTSEG, 2 * TOTSEG, 3 * TOTSEG
    BINLEN = 3 * TOTSEG + _NC * _NS * 16
    SHARE = BR // _NS
    EPCH = -(-(E // GC + SEGS) // 256) * 256
    EPAD = EPCH * GC
    assert E % _NS == 0 and EPT % S == 0 and S % 16 == 0 and SHARE % GC == 0
    assert NB % _NC == 0 and NBC <= 16

    mesh = plsc.VectorSubcoreMesh(
        core_axis_name="c", subcore_axis_name="s", num_cores=_NC, num_subcores=_NS
    )

    @functools.partial(
        pl.kernel,
        out_type=jax.ShapeDtypeStruct((BINLEN,), jnp.int32),
        mesh=mesh,
        scratch_types=[
            pltpu.VMEM((S,), jnp.int32),
            pltpu.VMEM((S,), jnp.int32),
            pltpu.VMEM((S,), jnp.int32),
            pltpu.VMEM((CAP,), jnp.int32),
            pltpu.VMEM((CAP,), jnp.int32),
            pltpu.VMEM((CAP,), jnp.int32),
            pltpu.VMEM((16,), jnp.int32),
            pltpu.SemaphoreType.DMA,
            pltpu.SemaphoreType.DMA,
        ],
        compiler_params=pltpu.CompilerParams(needs_layout_passes=False),
    )
    def sc_bin(src_hbm, dst_hbm, rel_hbm, out_hbm,
               dstv, relv, srcv, cl, cs, ce, cntv, si, sf):
        cid = lax.axis_index("c")
        sid = lax.axis_index("s")
        ebase = sid * EPT

        zero16i = jnp.zeros((16,), jnp.int32)
        trash16 = jnp.full((16,), TRASH, jnp.int32)
        iota16 = lax.iota(jnp.int32, 16)

        cntv[pl.ds(0, 16)] = zero16i

        @pl.loop(0, NBC)
        def _(bi):
            b = bi * _NC + cid
            lo = b * BR

            def chunk_body(ch, cnt):
                cbase = ebase + ch * S
                ld = pltpu.async_copy(dst_hbm.at[pl.ds(cbase, S)], dstv, si)
                lr = pltpu.async_copy(rel_hbm.at[pl.ds(cbase, S)], relv, si)
                ls = pltpu.async_copy(src_hbm.at[pl.ds(cbase, S)], srcv, si)
                ld.wait()
                lr.wait()
                ls.wait()

                def scan_body(v, cnt):
                    sl = pl.ds(v * 16, 16)
                    idx16 = dstv[sl] * R + relv[sl]
                    m = (idx16 >= lo) & (idx16 < lo + BR)
                    lidx16 = idx16 - lo
                    eid16 = cbase + v * 16 + iota16
                    mi = m.astype(jnp.int32)
                    pos16 = cnt + plsc.cumsum(mi) - 1
                    plsc.store_scatter(cl, [pos16], lidx16, mask=m)
                    plsc.store_scatter(cs, [pos16], srcv[sl], mask=m)
                    plsc.store_scatter(ce, [pos16], eid16, mask=m)
                    return cnt + jnp.sum(mi)

                return lax.fori_loop(0, S // 16, scan_body, cnt)

            cnt = lax.fori_loop(0, NCH, chunk_body, jnp.int32(0))

            @pl.loop(0, GC + 16, step=16)
            def _(i):
                cl[pl.ds(cnt + i, 16)] = trash16
                cs[pl.ds(cnt + i, 16)] = zero16i
                ce[pl.ds(cnt + i, 16)] = zero16i

            nfl = (cnt + GC - 1) // GC
            base = ((cid * _NS + sid) * NBC + bi) * CAP

            def flush_body(k, _):
                o = k * GC
                f1 = pltpu.async_copy(
                    cl.at[pl.ds(o, GC)], out_hbm.at[pl.ds(OFF_L + base + o, GC)], sf)
                f2 = pltpu.async_copy(
                    cs.at[pl.ds(o, GC)], out_hbm.at[pl.ds(OFF_S + base + o, GC)], sf)
                f3 = pltpu.async_copy(
                    ce.at[pl.ds(o, GC)], out_hbm.at[pl.ds(OFF_E + base + o, GC)], sf)
                f1.wait()
                f2.wait()
                f3.wait()
                return 0

            lax.fori_loop(0, nfl, flush_body, 0)
            plsc.store_scatter(
                cntv, [iota16], jnp.full((16,), nfl, jnp.int32), mask=(iota16 == bi))

        fc = pltpu.async_copy(
            cntv, out_hbm.at[pl.ds(OFF_C + (cid * _NS + sid) * 16, 16)], sf)
        fc.wait()

    def _seg_starts(bin_hbm, cv, st, sem, tid):
        lc = pltpu.async_copy(
            bin_hbm.at[pl.ds(OFF_C, _NC * _NS * 16)], cv.at[pl.ds(0, _NC * _NS * 16)],
            sem)
        lc.wait()

        def tb_body(t, s):
            v = cv[pl.ds(t * 16, 16)]
            return s + jnp.where(t < tid, jnp.sum(v), 0)

        tb = lax.fori_loop(0, _NC * _NS, tb_body, jnp.int32(0))
        own = cv[pl.ds(tid * 16, 16)]
        st[pl.ds(0, 16)] = tb + plsc.cumsum(own) - own
        st[pl.ds(16, 16)] = jnp.zeros((16,), jnp.int32)

    @functools.partial(
        pl.kernel,
        out_type=jax.ShapeDtypeStruct((EPAD, D), jnp.float32),
        mesh=mesh,
        scratch_types=[
            pltpu.VMEM((544,), jnp.int32),
            pltpu.VMEM((32,), jnp.int32),
            pltpu.VMEM((GC,), jnp.int32),
            pltpu.VMEM((GC,), jnp.int32),
            pltpu.VMEM((GC, D), jnp.float32),
            pltpu.VMEM((GC, D), jnp.float32),
            pltpu.SemaphoreType.DMA,
            pltpu.SemaphoreType.DMA,
            pltpu.SemaphoreType.DMA,
            pltpu.SemaphoreType.DMA,
        ],
        compiler_params=pltpu.CompilerParams(needs_layout_passes=False),
    )
    def sc_permute(he_in_hbm, bin_hbm, out_hbm,
                   cv, st, ce0, ce1, rw0, rw1, s0, s1, sf0, sf1):
        cid = lax.axis_index("c")
        sid = lax.axis_index("s")
        tid = cid * _NS + sid
        _seg_starts(bin_hbm, cv, st, s0, tid)

        @pl.loop(0, NBC)
        def _(bi):
            nfl = cv[pl.ds(tid * 16 + bi, 16)][0]
            start = st[pl.ds(bi, 16)][0]
            base = (tid * NBC + bi) * CAP
            npair = nfl // 2

            def pair_body(k, _):
                o0 = base + 2 * k * GC
                c0 = (start + 2 * k) * GC
                l0 = pltpu.async_copy(bin_hbm.at[pl.ds(OFF_E + o0, GC)], ce0, s0)
                l1 = pltpu.async_copy(bin_hbm.at[pl.ds(OFF_E + o0 + GC, GC)], ce1, s1)
                l0.wait()
                g0 = pltpu.async_copy(he_in_hbm.at[ce0], rw0, s0)
                l1.wait()
                g1 = pltpu.async_copy(he_in_hbm.at[ce1], rw1, s1)
                g0.wait()
                w0 = pltpu.async_copy(rw0, out_hbm.at[pl.ds(c0, GC)], sf0)
                g1.wait()
                w1 = pltpu.async_copy(rw1, out_hbm.at[pl.ds(c0 + GC, GC)], sf1)
                w0.wait()
                w1.wait()
                return 0

            lax.fori_loop(0, npair, pair_body, 0)

            @pl.when(nfl % 2 == 1)
            def _():
                o0 = base + npair * 2 * GC
                c0 = (start + npair * 2) * GC
                l0 = pltpu.async_copy(bin_hbm.at[pl.ds(OFF_E + o0, GC)], ce0, s0)
                l0.wait()
                g0 = pltpu.async_copy(he_in_hbm.at[ce0], rw0, s0)
                g0.wait()
                w0 = pltpu.async_copy(rw0, out_hbm.at[pl.ds(c0, GC)], sf0)
                w0.wait()

    @functools.partial(
        pl.kernel,
        out_type=jax.ShapeDtypeStruct((NB * BR, D), jnp.float32),
        mesh=mesh,
        scratch_types=[
            pltpu.VMEM((GC,), jnp.int32),
            pltpu.VMEM((GC,), jnp.int32),
            pltpu.VMEM((GC,), jnp.int32),
            pltpu.VMEM((GC,), jnp.int32),
            pltpu.VMEM((GC, D), jnp.float32),
            pltpu.VMEM((GC, D), jnp.float32),
            pltpu.VMEM((GC, D), jnp.float32),
            pltpu.VMEM((GC, D), jnp.float32),
            pltpu.VMEM((GC, D), jnp.float32),
            pltpu.VMEM((544,), jnp.int32),
            pltpu.VMEM((32,), jnp.int32),
            pltpu.VMEM_SHARED((BR + 8, D), jnp.float32),
            pltpu.SemaphoreType.DMA,
            pltpu.SemaphoreType.DMA,
            pltpu.SemaphoreType.DMA,
            pltpu.SemaphoreType.DMA,
            pltpu.SemaphoreType.DMA,
            pltpu.SemaphoreType.DMA,
        ],
        compiler_params=pltpu.CompilerParams(needs_layout_passes=False),
    )
    def sc_consume(hv_hbm, he_hbm, bin_hbm, out_hbm,
                   il0, is0, il1, is1, ra0, rb0, ra1, rb1, zb, cv, st,
                   acc, si0, si1, sg0, sg1, ss0, ss1):
        cid = lax.axis_index("c")
        sid = lax.axis_index("s")
        tid = cid * _NS + sid

        zero16f = jnp.zeros((16,), jnp.float32)

        _seg_starts(bin_hbm, cv, st, si0, tid)

        @pl.loop(0, GC)
        def _(r):
            for c in range(D // 16):
                zb[r, pl.ds(c * 16, 16)] = zero16f

        @pl.loop(0, NBC)
        def _(bi):
            b = bi * _NC + cid
            lo = b * BR

            for z in range(SHARE // GC):
                pltpu.sync_copy(zb, acc.at[pl.ds(sid * SHARE + z * GC, GC)])

            @pl.when(sid == 0)
            def _():
                pltpu.sync_copy(zb.at[pl.ds(0, 8)], acc.at[pl.ds(BR, 8)])

            plsc.subcore_barrier()

            n128 = cv[pl.ds(tid * 16 + bi, 16)][0]
            start = st[pl.ds(bi, 16)][0]
            base = (tid * NBC + bi) * CAP
            npairs = n128 // 2

            def pair_body(k, _):
                o0 = base + k * (2 * GC)
                o1 = o0 + GC
                c0 = (start + 2 * k) * GC
                i0a = pltpu.async_copy(bin_hbm.at[pl.ds(OFF_L + o0, GC)], il0, si0)
                i0b = pltpu.async_copy(bin_hbm.at[pl.ds(OFF_S + o0, GC)], is0, si0)
                h0 = pltpu.async_copy(he_hbm.at[pl.ds(c0, GC)], rb0, sg0)
                i1a = pltpu.async_copy(bin_hbm.at[pl.ds(OFF_L + o1, GC)], il1, si1)
                i1b = pltpu.async_copy(bin_hbm.at[pl.ds(OFF_S + o1, GC)], is1, si1)
                h1 = pltpu.async_copy(he_hbm.at[pl.ds(c0 + GC, GC)], rb1, sg1)
                i0a.wait()
                i0b.wait()
                g0 = pltpu.async_copy(hv_hbm.at[is0], ra0, sg0)
                i1a.wait()
                i1b.wait()
                g1 = pltpu.async_copy(hv_hbm.at[is1], ra1, sg1)
                g0.wait()
                h0.wait()
                s0a = pltpu.async_copy(ra0, acc.at[il0], ss0, add=True)
                s0b = pltpu.async_copy(rb0, acc.at[il0], ss0, add=True)
                g1.wait()
                h1.wait()
                s1a = pltpu.async_copy(ra1, acc.at[il1], ss1, add=True)
                s1b = pltpu.async_copy(rb1, acc.at[il1], ss1, add=True)
                s0a.wait()
                s0b.wait()
                s1a.wait()
                s1b.wait()
                return 0

            lax.fori_loop(0, npairs, pair_body, 0)

            @pl.when(n128 % 2 == 1)
            def _():
                o0 = base + npairs * (2 * GC)
                c0 = (start + npairs * 2) * GC
                ia = pltpu.async_copy(bin_hbm.at[pl.ds(OFF_L + o0, GC)], il0, si0)
                ib = pltpu.async_copy(bin_hbm.at[pl.ds(OFF_S + o0, GC)], is0, si0)
                hb = pltpu.async_copy(he_hbm.at[pl.ds(c0, GC)], rb0, sg0)
                ia.wait()
                ib.wait()
                ga = pltpu.async_copy(hv_hbm.at[is0], ra0, sg0)
                ga.wait()
                hb.wait()
                sa = pltpu.async_copy(ra0, acc.at[il0], ss0, add=True)
                sb = pltpu.async_copy(rb0, acc.at[il0], ss0, add=True)
                sa.wait()
                sb.wait()

            plsc.subcore_barrier()
            pltpu.sync_copy(
                acc.at[pl.ds(sid * SHARE, SHARE)],
                out_hbm.at[pl.ds(lo + sid * SHARE, SHARE)],
            )

    return sc_bin, sc_permute, sc_consume, EPAD


def kernel(h_v, edge_index, h_e, W_in, b_in, W_edge, b_edge, W_lin, W_out):
    N, D = h_v.shape
    E, DE = h_e.shape
    L = W_in.shape[0]
    R = W_lin.shape[1] // D
    NR = N * R

    sc_bin, sc_permute, sc_consume, EPAD = _make_sc_kernels(N, E, D, R)
    ei = edge_index.astype(jnp.int32)
    src, dst, rel = ei[0], ei[1], ei[2]

    binfo = sc_bin(src, dst, rel)
    h_e128 = jnp.pad(h_e, ((0, 0), (0, D - DE)))
    he_perm_in = sc_permute(h_e128, binfo)

    h = h_v
    for l in range(L):
        hv = _tc_in_mlp(h, W_in[l], b_in[l])
        he = _tc_edge_mlp(he_perm_in, W_edge[l], b_edge[l])
        upd_full = sc_consume(hv, he, binfo)
        upd = upd_full[:NR].reshape(N, R * D)
        h = _tc_out_mlp(upd, h, W_lin[l], W_out[l])
    return h


# speedup vs baseline: 1.0269x; 1.0269x over previous
import functools

import jax
import jax.numpy as jnp
from jax import lax
from jax.experimental import pallas as pl
from jax.experimental.pallas import tpu as pltpu
from jax.experimental.pallas import tpu_sc as plsc

_EPS = 1e-5
_BN = 1.0 / (1.0 + _EPS) ** 0.5


def _lrelu(x, slope):
    return jnp.where(x > 0, x, slope * x)


def _in_mlp_body(h_ref, w_ref, b_ref, o_ref):
    x = _lrelu(h_ref[...] * _BN, 0.2)
    y = jnp.dot(x, w_ref[...], preferred_element_type=jnp.float32) + b_ref[...]
    o_ref[...] = _lrelu(y * _BN, 0.2)


def _tc_in_mlp(h, W, b):
    N, D = h.shape
    BLK = 1000
    return pl.pallas_call(
        _in_mlp_body,
        grid=(N // BLK,),
        in_specs=[
            pl.BlockSpec((BLK, D), lambda i: (i, 0)),
            pl.BlockSpec((D, D), lambda i: (0, 0)),
            pl.BlockSpec((1, D), lambda i: (0, 0)),
        ],
        out_specs=pl.BlockSpec((BLK, D), lambda i: (i, 0)),
        out_shape=jax.ShapeDtypeStruct((N, D), jnp.float32),
    )(h, W, b.reshape(1, D))


def _edge_mlp_body(he_ref, w_ref, b_ref, o_ref):
    x = _lrelu(he_ref[...] * _BN, 0.2)[:, : w_ref.shape[0]]
    y = jnp.dot(x, w_ref[...], preferred_element_type=jnp.float32) + b_ref[...]
    o_ref[...] = _lrelu(y * _BN, 0.2)


def _tc_edge_mlp(h_e, W, b):
    E, _ = h_e.shape
    DE, D = W.shape
    BLK = 4096 if E % 4096 == 0 else 4000
    return pl.pallas_call(
        _edge_mlp_body,
        grid=(E // BLK,),
        in_specs=[
            pl.BlockSpec((BLK, h_e.shape[1]), lambda i: (i, 0)),
            pl.BlockSpec((DE, D), lambda i: (0, 0)),
            pl.BlockSpec((1, D), lambda i: (0, 0)),
        ],
        out_specs=pl.BlockSpec((BLK, D), lambda i: (i, 0)),
        out_shape=jax.ShapeDtypeStruct((E, D), jnp.float32),
    )(h_e, W, b.reshape(1, D))


def _out_mlp_body(u_ref, h_ref, wl_ref, wo_ref, o_ref):
    x = _lrelu(u_ref[...] * _BN, 0.1)
    y = jnp.dot(x, wl_ref[...], preferred_element_type=jnp.float32)
    y = _lrelu(y * _BN, 0.1)
    z = jnp.dot(y, wo_ref[...], preferred_element_type=jnp.float32)
    o_ref[...] = z + h_ref[...]


def _tc_out_mlp(upd, h, W_lin, W_out):
    N, RD = upd.shape
    D = W_out.shape[1]
    BLK = 1000
    return pl.pallas_call(
        _out_mlp_body,
        grid=(N // BLK,),
        in_specs=[
            pl.BlockSpec((BLK, RD), lambda i: (i, 0)),
            pl.BlockSpec((BLK, D), lambda i: (i, 0)),
            pl.BlockSpec((RD, D), lambda i: (0, 0)),
            pl.BlockSpec((D, D), lambda i: (0, 0)),
        ],
        out_specs=pl.BlockSpec((BLK, D), lambda i: (i, 0)),
        out_shape=jax.ShapeDtypeStruct((N, D), jnp.float32),
    )(upd, h, W_lin, W_out)


_NC, _NS = 2, 16


@functools.lru_cache(maxsize=None)
def _make_sc_kernels(N, E, D, R):
    NR = N * R
    BR = 4096
    NB = -(-NR // BR)
    NBC = NB // _NC
    TRASH = BR
    EPT = E // _NS
    S = 4000
    NCH = EPT // S
    GC = 128
    CAP = ((EPT + 143) // GC + 1) * GC
    SEGS = _NC * _NS * NBC
    TOTSEG = SEGS * CAP
    OFF_L, OFF_S, OFF_E, OFF_C = 0, TOTSEG, 2 * TOTSEG, 3 * TOTSEG
    BINLEN = 3 * TOTSEG + _NC * _NS * 16
    SHARE = BR // _NS
    EPCH = -(-(E // GC + SEGS) // 256) * 256
    EPAD = EPCH * GC
    assert E % _NS == 0 and EPT % S == 0 and S % 16 == 0 and SHARE % GC == 0
    assert NB % _NC == 0 and NBC <= 16

    mesh = plsc.VectorSubcoreMesh(
        core_axis_name="c", subcore_axis_name="s", num_cores=_NC, num_subcores=_NS
    )

    @functools.partial(
        pl.kernel,
        out_type=jax.ShapeDtypeStruct((BINLEN,), jnp.int32),
        mesh=mesh,
        scratch_types=[
            pltpu.VMEM((S,), jnp.int32),
            pltpu.VMEM((S,), jnp.int32),
            pltpu.VMEM((S,), jnp.int32),
            pltpu.VMEM((CAP,), jnp.int32),
            pltpu.VMEM((CAP,), jnp.int32),
            pltpu.VMEM((CAP,), jnp.int32),
            pltpu.VMEM((16,), jnp.int32),
            pltpu.SemaphoreType.DMA,
            pltpu.SemaphoreType.DMA,
        ],
        compiler_params=pltpu.CompilerParams(needs_layout_passes=False),
    )
    def sc_bin(src_hbm, dst_hbm, rel_hbm, out_hbm,
               dstv, relv, srcv, cl, cs, ce, cntv, si, sf):
        cid = lax.axis_index("c")
        sid = lax.axis_index("s")
        ebase = sid * EPT

        zero16i = jnp.zeros((16,), jnp.int32)
        trash16 = jnp.full((16,), TRASH, jnp.int32)
        iota16 = lax.iota(jnp.int32, 16)

        cntv[pl.ds(0, 16)] = zero16i

        @pl.loop(0, NBC)
        def _(bi):
            b = bi * _NC + cid
            lo = b * BR

            def chunk_body(ch, cnt):
                cbase = ebase + ch * S
                ld = pltpu.async_copy(dst_hbm.at[pl.ds(cbase, S)], dstv, si)
                lr = pltpu.async_copy(rel_hbm.at[pl.ds(cbase, S)], relv, si)
                ls = pltpu.async_copy(src_hbm.at[pl.ds(cbase, S)], srcv, si)
                ld.wait()
                lr.wait()
                ls.wait()

                def scan_body(v, cnt):
                    sl = pl.ds(v * 16, 16)
                    idx16 = dstv[sl] * R + relv[sl]
                    m = (idx16 >= lo) & (idx16 < lo + BR)
                    lidx16 = idx16 - lo
                    eid16 = cbase + v * 16 + iota16
                    mi = m.astype(jnp.int32)
                    pos16 = cnt + plsc.cumsum(mi) - 1
                    plsc.store_scatter(cl, [pos16], lidx16, mask=m)
                    plsc.store_scatter(cs, [pos16], srcv[sl], mask=m)
                    plsc.store_scatter(ce, [pos16], eid16, mask=m)
                    return cnt + jnp.sum(mi)

                return lax.fori_loop(0, S // 16, scan_body, cnt)

            cnt = lax.fori_loop(0, NCH, chunk_body, jnp.int32(0))

            @pl.loop(0, GC + 16, step=16)
            def _(i):
                cl[pl.ds(cnt + i, 16)] = trash16
                cs[pl.ds(cnt + i, 16)] = zero16i
                ce[pl.ds(cnt + i, 16)] = zero16i

            nfl = (cnt + GC - 1) // GC
            base = ((cid * _NS + sid) * NBC + bi) * CAP

            def flush_body(k, _):
                o = k * GC
                f1 = pltpu.async_copy(
                    cl.at[pl.ds(o, GC)], out_hbm.at[pl.ds(OFF_L + base + o, GC)], sf)
                f2 = pltpu.async_copy(
                    cs.at[pl.ds(o, GC)], out_hbm.at[pl.ds(OFF_S + base + o, GC)], sf)
                f3 = pltpu.async_copy(
                    ce.at[pl.ds(o, GC)], out_hbm.at[pl.ds(OFF_E + base + o, GC)], sf)
                f1.wait()
                f2.wait()
                f3.wait()
                return 0

            lax.fori_loop(0, nfl, flush_body, 0)
            plsc.store_scatter(
                cntv, [iota16], jnp.full((16,), nfl, jnp.int32), mask=(iota16 == bi))

        fc = pltpu.async_copy(
            cntv, out_hbm.at[pl.ds(OFF_C + (cid * _NS + sid) * 16, 16)], sf)
        fc.wait()

    def _seg_starts(bin_hbm, cv, st, sem, tid):
        lc = pltpu.async_copy(
            bin_hbm.at[pl.ds(OFF_C, _NC * _NS * 16)], cv.at[pl.ds(0, _NC * _NS * 16)],
            sem)
        lc.wait()

        def tb_body(t, s):
            v = cv[pl.ds(t * 16, 16)]
            return s + jnp.where(t < tid, jnp.sum(v), 0)

        tb = lax.fori_loop(0, _NC * _NS, tb_body, jnp.int32(0))
        own = cv[pl.ds(tid * 16, 16)]
        st[pl.ds(0, 16)] = tb + plsc.cumsum(own) - own
        st[pl.ds(16, 16)] = jnp.zeros((16,), jnp.int32)

    @functools.partial(
        pl.kernel,
        out_type=jax.ShapeDtypeStruct((EPAD, D), jnp.float32),
        mesh=mesh,
        scratch_types=[
            pltpu.VMEM((544,), jnp.int32),
            pltpu.VMEM((32,), jnp.int32),
            pltpu.VMEM((GC,), jnp.int32),
            pltpu.VMEM((GC,), jnp.int32),
            pltpu.VMEM((GC, D), jnp.float32),
            pltpu.VMEM((GC, D), jnp.float32),
            pltpu.SemaphoreType.DMA,
            pltpu.SemaphoreType.DMA,
            pltpu.SemaphoreType.DMA,
            pltpu.SemaphoreType.DMA,
        ],
        compiler_params=pltpu.CompilerParams(needs_layout_passes=False),
    )
    def sc_permute(he_in_hbm, bin_hbm, out_hbm,
                   cv, st, ce0, ce1, rw0, rw1, s0, s1, sf0, sf1):
        cid = lax.axis_index("c")
        sid = lax.axis_index("s")
        tid = cid * _NS + sid
        _seg_starts(bin_hbm, cv, st, s0, tid)

        @pl.loop(0, NBC)
        def _(bi):
            nfl = cv[pl.ds(tid * 16 + bi, 16)][0]
            start = st[pl.ds(bi, 16)][0]
            base = (tid * NBC + bi) * CAP
            npair = nfl // 2

            def pair_body(k, _):
                o0 = base + 2 * k * GC
                c0 = (start + 2 * k) * GC
                l0 = pltpu.async_copy(bin_hbm.at[pl.ds(OFF_E + o0, GC)], ce0, s0)
                l1 = pltpu.async_copy(bin_hbm.at[pl.ds(OFF_E + o0 + GC, GC)], ce1, s1)
                l0.wait()
                g0 = pltpu.async_copy(he_in_hbm.at[ce0], rw0, s0)
                l1.wait()
                g1 = pltpu.async_copy(he_in_hbm.at[ce1], rw1, s1)
                g0.wait()
                w0 = pltpu.async_copy(rw0, out_hbm.at[pl.ds(c0, GC)], sf0)
                g1.wait()
                w1 = pltpu.async_copy(rw1, out_hbm.at[pl.ds(c0 + GC, GC)], sf1)
                w0.wait()
                w1.wait()
                return 0

            lax.fori_loop(0, npair, pair_body, 0)

            @pl.when(nfl % 2 == 1)
            def _():
                o0 = base + npair * 2 * GC
                c0 = (start + npair * 2) * GC
                l0 = pltpu.async_copy(bin_hbm.at[pl.ds(OFF_E + o0, GC)], ce0, s0)
                l0.wait()
                g0 = pltpu.async_copy(he_in_hbm.at[ce0], rw0, s0)
                g0.wait()
                w0 = pltpu.async_copy(rw0, out_hbm.at[pl.ds(c0, GC)], sf0)
                w0.wait()

    @functools.partial(
        pl.kernel,
        out_type=jax.ShapeDtypeStruct((NB * BR, D), jnp.float32),
        mesh=mesh,
        scratch_types=[
            pltpu.VMEM((GC,), jnp.int32),
            pltpu.VMEM((GC,), jnp.int32),
            pltpu.VMEM((GC,), jnp.int32),
            pltpu.VMEM((GC,), jnp.int32),
            pltpu.VMEM((GC, D), jnp.float32),
            pltpu.VMEM((GC, D), jnp.float32),
            pltpu.VMEM((GC, D), jnp.float32),
            pltpu.VMEM((544,), jnp.int32),
            pltpu.VMEM((32,), jnp.int32),
            pltpu.VMEM_SHARED((BR + 8, D), jnp.float32),
            pltpu.SemaphoreType.DMA,
            pltpu.SemaphoreType.DMA,
            pltpu.SemaphoreType.DMA,
            pltpu.SemaphoreType.DMA,
            pltpu.SemaphoreType.DMA,
            pltpu.SemaphoreType.DMA,
        ],
        compiler_params=pltpu.CompilerParams(needs_layout_passes=False),
    )
    def sc_consume(hv_hbm, he_hbm, bin_hbm, out_hbm,
                   il0, is0, il1, is1, rb0, rb1, zb, cv, st,
                   acc, si0, si1, sg0, sg1, ss0, ss1):
        cid = lax.axis_index("c")
        sid = lax.axis_index("s")
        tid = cid * _NS + sid

        zero16f = jnp.zeros((16,), jnp.float32)

        _seg_starts(bin_hbm, cv, st, si0, tid)

        @pl.loop(0, GC)
        def _(r):
            for c in range(D // 16):
                zb[r, pl.ds(c * 16, 16)] = zero16f

        @pl.loop(0, NBC)
        def _(bi):
            b = bi * _NC + cid
            lo = b * BR

            for z in range(SHARE // GC):
                pltpu.sync_copy(zb, acc.at[pl.ds(sid * SHARE + z * GC, GC)])

            @pl.when(sid == 0)
            def _():
                pltpu.sync_copy(zb.at[pl.ds(0, 8)], acc.at[pl.ds(BR, 8)])

            plsc.subcore_barrier()

            n128 = cv[pl.ds(tid * 16 + bi, 16)][0]
            start = st[pl.ds(bi, 16)][0]
            base = (tid * NBC + bi) * CAP
            npairs = n128 // 2

            def pair_body(k, _):
                o0 = base + k * (2 * GC)
                o1 = o0 + GC
                c0 = (start + 2 * k) * GC
                i0a = pltpu.async_copy(bin_hbm.at[pl.ds(OFF_L + o0, GC)], il0, si0)
                i0b = pltpu.async_copy(bin_hbm.at[pl.ds(OFF_S + o0, GC)], is0, si0)
                h0 = pltpu.async_copy(he_hbm.at[pl.ds(c0, GC)], rb0, sg0)
                i1a = pltpu.async_copy(bin_hbm.at[pl.ds(OFF_L + o1, GC)], il1, si1)
                i1b = pltpu.async_copy(bin_hbm.at[pl.ds(OFF_S + o1, GC)], is1, si1)
                h1 = pltpu.async_copy(he_hbm.at[pl.ds(c0 + GC, GC)], rb1, sg1)
                i0a.wait()
                i0b.wait()
                h0.wait()
                g0 = pltpu.async_copy(hv_hbm.at[is0], rb0, sg0, add=True)
                i1a.wait()
                i1b.wait()
                h1.wait()
                g1 = pltpu.async_copy(hv_hbm.at[is1], rb1, sg1, add=True)
                g0.wait()
                s0 = pltpu.async_copy(rb0, acc.at[il0], ss0, add=True)
                g1.wait()
                s1 = pltpu.async_copy(rb1, acc.at[il1], ss1, add=True)
                s0.wait()
                s1.wait()
                return 0

            lax.fori_loop(0, npairs, pair_body, 0)

            @pl.when(n128 % 2 == 1)
            def _():
                o0 = base + npairs * (2 * GC)
                c0 = (start + npairs * 2) * GC
                ia = pltpu.async_copy(bin_hbm.at[pl.ds(OFF_L + o0, GC)], il0, si0)
                ib = pltpu.async_copy(bin_hbm.at[pl.ds(OFF_S + o0, GC)], is0, si0)
                hb = pltpu.async_copy(he_hbm.at[pl.ds(c0, GC)], rb0, sg0)
                ia.wait()
                ib.wait()
                hb.wait()
                ga = pltpu.async_copy(hv_hbm.at[is0], rb0, sg0, add=True)
                ga.wait()
                sa = pltpu.async_copy(rb0, acc.at[il0], ss0, add=True)
                sa.wait()

            plsc.subcore_barrier()
            pltpu.sync_copy(
                acc.at[pl.ds(sid * SHARE, SHARE)],
                out_hbm.at[pl.ds(lo + sid * SHARE, SHARE)],
            )

    return sc_bin, sc_permute, sc_consume, EPAD


def kernel(h_v, edge_index, h_e, W_in, b_in, W_edge, b_edge, W_lin, W_out):
    N, D = h_v.shape
    E, DE = h_e.shape
    L = W_in.shape[0]
    R = W_lin.shape[1] // D
    NR = N * R

    sc_bin, sc_permute, sc_consume, EPAD = _make_sc_kernels(N, E, D, R)
    ei = edge_index.astype(jnp.int32)
    src, dst, rel = ei[0], ei[1], ei[2]

    binfo = sc_bin(src, dst, rel)
    h_e128 = jnp.pad(h_e, ((0, 0), (0, D - DE)))
    he_perm_in = sc_permute(h_e128, binfo)

    h = h_v
    for l in range(L):
        hv = _tc_in_mlp(h, W_in[l], b_in[l])
        he = _tc_edge_mlp(he_perm_in, W_edge[l], b_edge[l])
        upd_full = sc_consume(hv, he, binfo)
        upd = upd_full[:NR].reshape(N, R * D)
        h = _tc_out_mlp(upd, h, W_lin[l], W_out[l])
    return h


# speedup vs baseline: 1.0375x; 1.0103x over previous
import functools

import jax
import jax.numpy as jnp
from jax import lax
from jax.experimental import pallas as pl
from jax.experimental.pallas import tpu as pltpu
from jax.experimental.pallas import tpu_sc as plsc

_EPS = 1e-5
_BN = 1.0 / (1.0 + _EPS) ** 0.5


def _lrelu(x, slope):
    return jnp.where(x > 0, x, slope * x)


def _in_mlp_body(h_ref, w_ref, b_ref, o_ref):
    x = _lrelu(h_ref[...] * _BN, 0.2)
    y = jnp.dot(x, w_ref[...], preferred_element_type=jnp.float32) + b_ref[...]
    o_ref[...] = _lrelu(y * _BN, 0.2)


def _tc_in_mlp(h, W, b):
    N, D = h.shape
    BLK = 1000
    return pl.pallas_call(
        _in_mlp_body,
        grid=(N // BLK,),
        in_specs=[
            pl.BlockSpec((BLK, D), lambda i: (i, 0)),
            pl.BlockSpec((D, D), lambda i: (0, 0)),
            pl.BlockSpec((1, D), lambda i: (0, 0)),
        ],
        out_specs=pl.BlockSpec((BLK, D), lambda i: (i, 0)),
        out_shape=jax.ShapeDtypeStruct((N, D), jnp.float32),
    )(h, W, b.reshape(1, D))


def _edge_mlp_body(he_ref, w_ref, b_ref, o_ref):
    x = _lrelu(he_ref[...] * _BN, 0.2)[:, : w_ref.shape[0]]
    y = jnp.dot(x, w_ref[...], preferred_element_type=jnp.float32) + b_ref[...]
    o_ref[...] = _lrelu(y * _BN, 0.2)


def _tc_edge_mlp(h_e, W, b):
    E, _ = h_e.shape
    DE, D = W.shape
    BLK = 4096 if E % 4096 == 0 else 4000
    return pl.pallas_call(
        _edge_mlp_body,
        grid=(E // BLK,),
        in_specs=[
            pl.BlockSpec((BLK, h_e.shape[1]), lambda i: (i, 0)),
            pl.BlockSpec((DE, D), lambda i: (0, 0)),
            pl.BlockSpec((1, D), lambda i: (0, 0)),
        ],
        out_specs=pl.BlockSpec((BLK, D), lambda i: (i, 0)),
        out_shape=jax.ShapeDtypeStruct((E, D), jnp.float32),
    )(h_e, W, b.reshape(1, D))


def _out_mlp_body(u_ref, h_ref, wl_ref, wo_ref, o_ref):
    x = _lrelu(u_ref[...] * _BN, 0.1)
    y = jnp.dot(x, wl_ref[...], preferred_element_type=jnp.float32)
    y = _lrelu(y * _BN, 0.1)
    z = jnp.dot(y, wo_ref[...], preferred_element_type=jnp.float32)
    o_ref[...] = z + h_ref[...]


def _tc_out_mlp(upd, h, W_lin, W_out):
    N, RD = upd.shape
    D = W_out.shape[1]
    BLK = 1000
    return pl.pallas_call(
        _out_mlp_body,
        grid=(N // BLK,),
        in_specs=[
            pl.BlockSpec((BLK, RD), lambda i: (i, 0)),
            pl.BlockSpec((BLK, D), lambda i: (i, 0)),
            pl.BlockSpec((RD, D), lambda i: (0, 0)),
            pl.BlockSpec((D, D), lambda i: (0, 0)),
        ],
        out_specs=pl.BlockSpec((BLK, D), lambda i: (i, 0)),
        out_shape=jax.ShapeDtypeStruct((N, D), jnp.float32),
    )(upd, h, W_lin, W_out)


_NC, _NS = 2, 16


@functools.lru_cache(maxsize=None)
def _make_sc_kernels(N, E, D, R):
    NR = N * R
    BR = 4096
    NB = -(-NR // BR)
    NBC = NB // _NC
    TRASH = BR
    EPT = E // _NS
    S = 4000
    NCH = EPT // S
    GC = 128
    CAP = ((EPT + 143) // GC + 1) * GC
    SEGS = _NC * _NS * NBC
    TOTSEG = SEGS * CAP
    OFF_L, OFF_S, OFF_E, OFF_C = 0, TOTSEG, 2 * TOTSEG, 3 * TOTSEG
    BINLEN = 3 * TOTSEG + _NC * _NS * 16
    SHARE = BR // _NS
    EPCH = -(-(E // GC + SEGS) // 256) * 256
    EPAD = EPCH * GC
    assert E % _NS == 0 and EPT % S == 0 and S % 16 == 0 and SHARE % GC == 0
    assert NB % _NC == 0 and NBC <= 16

    mesh = plsc.VectorSubcoreMesh(
        core_axis_name="c", subcore_axis_name="s", num_cores=_NC, num_subcores=_NS
    )

    @functools.partial(
        pl.kernel,
        out_type=jax.ShapeDtypeStruct((BINLEN,), jnp.int32),
        mesh=mesh,
        scratch_types=[
            pltpu.VMEM((S,), jnp.int32),
            pltpu.VMEM((S,), jnp.int32),
            pltpu.VMEM((S,), jnp.int32),
            pltpu.VMEM((CAP,), jnp.int32),
            pltpu.VMEM((CAP,), jnp.int32),
            pltpu.VMEM((CAP,), jnp.int32),
            pltpu.VMEM((16,), jnp.int32),
            pltpu.SemaphoreType.DMA,
            pltpu.SemaphoreType.DMA,
        ],
        compiler_params=pltpu.CompilerParams(needs_layout_passes=False),
    )
    def sc_bin(src_hbm, dst_hbm, rel_hbm, out_hbm,
               dstv, relv, srcv, cl, cs, ce, cntv, si, sf):
        cid = lax.axis_index("c")
        sid = lax.axis_index("s")
        ebase = sid * EPT

        zero16i = jnp.zeros((16,), jnp.int32)
        trash16 = jnp.full((16,), TRASH, jnp.int32)
        iota16 = lax.iota(jnp.int32, 16)

        cntv[pl.ds(0, 16)] = zero16i

        @pl.loop(0, NBC)
        def _(bi):
            b = bi * _NC + cid
            lo = b * BR

            def chunk_body(ch, cnt):
                cbase = ebase + ch * S
                ld = pltpu.async_copy(dst_hbm.at[pl.ds(cbase, S)], dstv, si)
                lr = pltpu.async_copy(rel_hbm.at[pl.ds(cbase, S)], relv, si)
                ls = pltpu.async_copy(src_hbm.at[pl.ds(cbase, S)], srcv, si)
                ld.wait()
                lr.wait()
                ls.wait()

                def scan_body(v, cnt):
                    sl = pl.ds(v * 16, 16)
                    idx16 = dstv[sl] * R + relv[sl]
                    m = (idx16 >= lo) & (idx16 < lo + BR)
                    lidx16 = idx16 - lo
                    eid16 = cbase + v * 16 + iota16
                    mi = m.astype(jnp.int32)
                    pos16 = cnt + plsc.cumsum(mi) - 1
                    plsc.store_scatter(cl, [pos16], lidx16, mask=m)
                    plsc.store_scatter(cs, [pos16], srcv[sl], mask=m)
                    plsc.store_scatter(ce, [pos16], eid16, mask=m)
                    return cnt + jnp.sum(mi)

                return lax.fori_loop(0, S // 16, scan_body, cnt)

            cnt = lax.fori_loop(0, NCH, chunk_body, jnp.int32(0))

            @pl.loop(0, GC + 16, step=16)
            def _(i):
                cl[pl.ds(cnt + i, 16)] = trash16
                cs[pl.ds(cnt + i, 16)] = zero16i
                ce[pl.ds(cnt + i, 16)] = zero16i

            nfl = (cnt + GC - 1) // GC
            base = ((cid * _NS + sid) * NBC + bi) * CAP

            def flush_body(k, _):
                o = k * GC
                f1 = pltpu.async_copy(
                    cl.at[pl.ds(o, GC)], out_hbm.at[pl.ds(OFF_L + base + o, GC)], sf)
                f2 = pltpu.async_copy(
                    cs.at[pl.ds(o, GC)], out_hbm.at[pl.ds(OFF_S + base + o, GC)], sf)
                f3 = pltpu.async_copy(
                    ce.at[pl.ds(o, GC)], out_hbm.at[pl.ds(OFF_E + base + o, GC)], sf)
                f1.wait()
                f2.wait()
                f3.wait()
                return 0

            lax.fori_loop(0, nfl, flush_body, 0)
            plsc.store_scatter(
                cntv, [iota16], jnp.full((16,), nfl, jnp.int32), mask=(iota16 == bi))

        fc = pltpu.async_copy(
            cntv, out_hbm.at[pl.ds(OFF_C + (cid * _NS + sid) * 16, 16)], sf)
        fc.wait()

    def _seg_starts(bin_hbm, cv, st, sem, tid):
        lc = pltpu.async_copy(
            bin_hbm.at[pl.ds(OFF_C, _NC * _NS * 16)], cv.at[pl.ds(0, _NC * _NS * 16)],
            sem)
        lc.wait()

        def tb_body(t, s):
            v = cv[pl.ds(t * 16, 16)]
            return s + jnp.where(t < tid, jnp.sum(v), 0)

        tb = lax.fori_loop(0, _NC * _NS, tb_body, jnp.int32(0))
        own = cv[pl.ds(tid * 16, 16)]
        st[pl.ds(0, 16)] = tb + plsc.cumsum(own) - own
        st[pl.ds(16, 16)] = jnp.zeros((16,), jnp.int32)

    @functools.partial(
        pl.kernel,
        out_type=jax.ShapeDtypeStruct((EPAD, D), jnp.float32),
        mesh=mesh,
        scratch_types=[
            pltpu.VMEM((544,), jnp.int32),
            pltpu.VMEM((32,), jnp.int32),
            pltpu.VMEM((GC,), jnp.int32),
            pltpu.VMEM((GC,), jnp.int32),
            pltpu.VMEM((GC, D), jnp.float32),
            pltpu.VMEM((GC, D), jnp.float32),
            pltpu.SemaphoreType.DMA,
            pltpu.SemaphoreType.DMA,
            pltpu.SemaphoreType.DMA,
            pltpu.SemaphoreType.DMA,
        ],
        compiler_params=pltpu.CompilerParams(needs_layout_passes=False),
    )
    def sc_permute(he_in_hbm, bin_hbm, out_hbm,
                   cv, st, ce0, ce1, rw0, rw1, s0, s1, sf0, sf1):
        cid = lax.axis_index("c")
        sid = lax.axis_index("s")
        tid = cid * _NS + sid
        _seg_starts(bin_hbm, cv, st, s0, tid)

        @pl.loop(0, NBC)
        def _(bi):
            nfl = cv[pl.ds(tid * 16 + bi, 16)][0]
            start = st[pl.ds(bi, 16)][0]
            base = (tid * NBC + bi) * CAP
            npair = nfl // 2

            def pair_body(k, _):
                o0 = base + 2 * k * GC
                c0 = (start + 2 * k) * GC
                l0 = pltpu.async_copy(bin_hbm.at[pl.ds(OFF_E + o0, GC)], ce0, s0)
                l1 = pltpu.async_copy(bin_hbm.at[pl.ds(OFF_E + o0 + GC, GC)], ce1, s1)
                l0.wait()
                g0 = pltpu.async_copy(he_in_hbm.at[ce0], rw0, s0)
                l1.wait()
                g1 = pltpu.async_copy(he_in_hbm.at[ce1], rw1, s1)
                g0.wait()
                w0 = pltpu.async_copy(rw0, out_hbm.at[pl.ds(c0, GC)], sf0)
                g1.wait()
                w1 = pltpu.async_copy(rw1, out_hbm.at[pl.ds(c0 + GC, GC)], sf1)
                w0.wait()
                w1.wait()
                return 0

            lax.fori_loop(0, npair, pair_body, 0)

            @pl.when(nfl % 2 == 1)
            def _():
                o0 = base + npair * 2 * GC
                c0 = (start + npair * 2) * GC
                l0 = pltpu.async_copy(bin_hbm.at[pl.ds(OFF_E + o0, GC)], ce0, s0)
                l0.wait()
                g0 = pltpu.async_copy(he_in_hbm.at[ce0], rw0, s0)
                g0.wait()
                w0 = pltpu.async_copy(rw0, out_hbm.at[pl.ds(c0, GC)], sf0)
                w0.wait()

    @functools.partial(
        pl.kernel,
        out_type=jax.ShapeDtypeStruct((NB * BR, D), jnp.float32),
        mesh=mesh,
        scratch_types=[
            pltpu.VMEM((GC,), jnp.int32),
            pltpu.VMEM((GC,), jnp.int32),
            pltpu.VMEM((GC,), jnp.int32),
            pltpu.VMEM((GC,), jnp.int32),
            pltpu.VMEM((GC,), jnp.int32),
            pltpu.VMEM((GC,), jnp.int32),
            pltpu.VMEM((GC,), jnp.int32),
            pltpu.VMEM((GC,), jnp.int32),
            pltpu.VMEM((GC, D), jnp.float32),
            pltpu.VMEM((GC, D), jnp.float32),
            pltpu.VMEM((GC, D), jnp.float32),
            pltpu.VMEM((GC, D), jnp.float32),
            pltpu.VMEM((GC, D), jnp.float32),
            pltpu.VMEM((544,), jnp.int32),
            pltpu.VMEM((32,), jnp.int32),
            pltpu.VMEM_SHARED((BR + 8, D), jnp.float32),
            pltpu.SemaphoreType.DMA,
            pltpu.SemaphoreType.DMA,
            pltpu.SemaphoreType.DMA,
            pltpu.SemaphoreType.DMA,
            pltpu.SemaphoreType.DMA,
            pltpu.SemaphoreType.DMA,
        ],
        compiler_params=pltpu.CompilerParams(needs_layout_passes=False),
    )
    def sc_consume(hv_hbm, he_hbm, bin_hbm, out_hbm,
                   il0, is0, il1, is1, il2, is2, il3, is3,
                   rb0, rb1, rb2, rb3, zb, cv, st,
                   acc, si0, si1, sg0, sg1, ss0, ss1):
        cid = lax.axis_index("c")
        sid = lax.axis_index("s")
        tid = cid * _NS + sid

        zero16f = jnp.zeros((16,), jnp.float32)

        _seg_starts(bin_hbm, cv, st, si0, tid)

        @pl.loop(0, GC)
        def _(r):
            for c in range(D // 16):
                zb[r, pl.ds(c * 16, 16)] = zero16f

        ils = [il0, il1, il2, il3]
        iss = [is0, is1, is2, is3]
        rbs = [rb0, rb1, rb2, rb3]
        sis = [si0, si1, si0, si1]
        sgs = [sg0, sg1, sg0, sg1]
        sss = [ss0, ss1, ss0, ss1]

        @pl.loop(0, NBC)
        def _(bi):
            b = bi * _NC + cid
            lo = b * BR

            for z in range(SHARE // GC):
                pltpu.sync_copy(zb, acc.at[pl.ds(sid * SHARE + z * GC, GC)])

            @pl.when(sid == 0)
            def _():
                pltpu.sync_copy(zb.at[pl.ds(0, 8)], acc.at[pl.ds(BR, 8)])

            plsc.subcore_barrier()

            n128 = cv[pl.ds(tid * 16 + bi, 16)][0]
            start = st[pl.ds(bi, 16)][0]
            base = (tid * NBC + bi) * CAP
            nquads = n128 // 4

            def quad_body(k, _):
                q0 = 4 * k
                idma = []
                hdma = []
                for j in range(4):
                    o = base + (q0 + j) * GC
                    c = (start + q0 + j) * GC
                    ia = pltpu.async_copy(
                        bin_hbm.at[pl.ds(OFF_L + o, GC)], ils[j], sis[j])
                    ib = pltpu.async_copy(
                        bin_hbm.at[pl.ds(OFF_S + o, GC)], iss[j], sis[j])
                    h = pltpu.async_copy(he_hbm.at[pl.ds(c, GC)], rbs[j], sgs[j])
                    idma.append((ia, ib))
                    hdma.append(h)
                gdma = []
                for j in range(4):
                    idma[j][0].wait()
                    idma[j][1].wait()
                    hdma[j].wait()
                    gdma.append(
                        pltpu.async_copy(hv_hbm.at[iss[j]], rbs[j], sgs[j],
                                         add=True))
                sdma = []
                for j in range(4):
                    gdma[j].wait()
                    sdma.append(
                        pltpu.async_copy(rbs[j], acc.at[ils[j]], sss[j],
                                         add=True))
                for j in range(4):
                    sdma[j].wait()
                return 0

            lax.fori_loop(0, nquads, quad_body, 0)

            def tail_body(ch, _):
                o = base + ch * GC
                c = (start + ch) * GC
                ia = pltpu.async_copy(bin_hbm.at[pl.ds(OFF_L + o, GC)], il0, si0)
                ib = pltpu.async_copy(bin_hbm.at[pl.ds(OFF_S + o, GC)], is0, si0)
                hb = pltpu.async_copy(he_hbm.at[pl.ds(c, GC)], rb0, sg0)
                ia.wait()
                ib.wait()
                hb.wait()
                ga = pltpu.async_copy(hv_hbm.at[is0], rb0, sg0, add=True)
                ga.wait()
                sa = pltpu.async_copy(rb0, acc.at[il0], ss0, add=True)
                sa.wait()
                return 0

            lax.fori_loop(nquads * 4, n128, tail_body, 0)

            plsc.subcore_barrier()
            pltpu.sync_copy(
                acc.at[pl.ds(sid * SHARE, SHARE)],
                out_hbm.at[pl.ds(lo + sid * SHARE, SHARE)],
            )

    return sc_bin, sc_permute, sc_consume, EPAD


def kernel(h_v, edge_index, h_e, W_in, b_in, W_edge, b_edge, W_lin, W_out):
    N, D = h_v.shape
    E, DE = h_e.shape
    L = W_in.shape[0]
    R = W_lin.shape[1] // D
    NR = N * R

    sc_bin, sc_permute, sc_consume, EPAD = _make_sc_kernels(N, E, D, R)
    ei = edge_index.astype(jnp.int32)
    src, dst, rel = ei[0], ei[1], ei[2]

    binfo = sc_bin(src, dst, rel)
    h_e128 = jnp.pad(h_e, ((0, 0), (0, D - DE)))
    he_perm_in = sc_permute(h_e128, binfo)

    h = h_v
    for l in range(L):
        hv = _tc_in_mlp(h, W_in[l], b_in[l])
        he = _tc_edge_mlp(he_perm_in, W_edge[l], b_edge[l])
        upd_full = sc_consume(hv, he, binfo)
        upd = upd_full[:NR].reshape(N, R * D)
        h = _tc_out_mlp(upd, h, W_lin[l], W_out[l])
    return h
